# initial kernel scaffold (unmeasured)
import jax
import jax.numpy as jnp
from jax import lax
from jax.experimental import pallas as pl
from jax.experimental.pallas import tpu as pltpu


def kernel(x, W):
    t, d = x.shape
    _, v_loc = W.shape
    v_tot = 2 * v_loc

    def body(x_ref, w_ref, out_ref, send_ref, recv_ref, send_sem, recv_sem):
        my_x = lax.axis_index("x")
        my_y = lax.axis_index("y")
        my_z = lax.axis_index("z")
        peer = (1 - my_x, my_y, my_z)

        barrier_sem = pltpu.get_barrier_semaphore()
        pl.semaphore_signal(
            barrier_sem,
            inc=1,
            device_id=peer,
            device_id_type=pl.DeviceIdType.MESH,
        )
        pl.semaphore_wait(barrier_sem, 1)

        send_ref[:, :] = jnp.dot(
            x_ref[:, :], w_ref[:, :], preferred_element_type=jnp.float32
        )

        rdma = pltpu.make_async_remote_copy(
            src_ref=send_ref,
            dst_ref=recv_ref,
            send_sem=send_sem,
            recv_sem=recv_sem,
            device_id=peer,
            device_id_type=pl.DeviceIdType.MESH,
        )
        rdma.start()
        rdma.wait()

        @pl.when(my_x == 0)
        def _():
            out_ref[:, :v_loc] = send_ref[:, :]
            out_ref[:, v_loc:] = recv_ref[:, :]

        @pl.when(my_x == 1)
        def _():
            out_ref[:, :v_loc] = recv_ref[:, :]
            out_ref[:, v_loc:] = send_ref[:, :]

        full = out_ref[:, :]
        m = jnp.max(full, axis=1, keepdims=True)
        e = jnp.exp(full - m)
        s = jnp.sum(e, axis=1, keepdims=True)
        out_ref[:, :] = e / s

    return pl.pallas_call(
        body,
        out_shape=jax.ShapeDtypeStruct((t, v_tot), jnp.float32),
        in_specs=[
            pl.BlockSpec(memory_space=pltpu.VMEM),
            pl.BlockSpec(memory_space=pltpu.VMEM),
        ],
        out_specs=pl.BlockSpec(memory_space=pltpu.VMEM),
        scratch_shapes=[
            pltpu.VMEM((t, v_loc), jnp.float32),
            pltpu.VMEM((t, v_loc), jnp.float32),
            pltpu.SemaphoreType.DMA,
            pltpu.SemaphoreType.DMA,
        ],
        compiler_params=pltpu.CompilerParams(collective_id=0),
    )(x, W)


# baseline (device time: 250249 ns/iter reference)
import jax
import jax.numpy as jnp
from jax import lax
from jax.experimental import pallas as pl
from jax.experimental.pallas import tpu as pltpu

W_CHUNK = 512
ROW_BLK = 32


def kernel(x, W):
    t, d = x.shape
    _, v_loc = W.shape
    v_tot = 2 * v_loc
    n_chunks = v_loc // W_CHUNK

    def body(x_ref, w_hbm, out_ref, wbuf, wsems, send_sem, recv_sem):
        my_x = lax.axis_index("x")
        my_y = lax.axis_index("y")
        my_z = lax.axis_index("z")
        peer = (1 - my_x, my_y, my_z)

        barrier_sem = pltpu.get_barrier_semaphore()
        pl.semaphore_signal(
            barrier_sem,
            inc=1,
            device_id=peer,
            device_id_type=pl.DeviceIdType.MESH,
        )
        pl.semaphore_wait(barrier_sem, 1)

        def gemm_and_exchange(xpos):
            base = xpos * v_loc
            def fetch(j, slot):
                return pltpu.make_async_copy(
                    w_hbm.at[:, pl.ds(j * W_CHUNK, W_CHUNK)],
                    wbuf.at[slot],
                    wsems.at[slot],
                )

            fetch(0, 0).start()

            def gemm_step(j, carry):
                slot = j % 2

                @pl.when(j + 1 < n_chunks)
                def _():
                    fetch(j + 1, (j + 1) % 2).start()

                fetch(j, slot).wait()
                out_ref[:, pl.ds(base + j * W_CHUNK, W_CHUNK)] = jnp.dot(
                    x_ref[:, :], wbuf[slot], preferred_element_type=jnp.float32
                )
                return carry

            lax.fori_loop(0, n_chunks, gemm_step, 0)

            rdma = pltpu.make_async_remote_copy(
                src_ref=out_ref.at[:, pl.ds(base, v_loc)],
                dst_ref=out_ref.at[:, pl.ds(base, v_loc)],
                send_sem=send_sem,
                recv_sem=recv_sem,
                device_id=peer,
                device_id_type=pl.DeviceIdType.MESH,
            )
            rdma.start()
            rdma.wait()

        @pl.when(my_x == 0)
        def _():
            gemm_and_exchange(0)

        @pl.when(my_x == 1)
        def _():
            gemm_and_exchange(1)

        def sm_step(i, carry):
            r = i * ROW_BLK
            blk = out_ref[pl.ds(r, ROW_BLK), :]
            m = jnp.max(blk, axis=1, keepdims=True)
            e = jnp.exp(blk - m)
            s = jnp.sum(e, axis=1, keepdims=True)
            out_ref[pl.ds(r, ROW_BLK), :] = e / s
            return carry

        lax.fori_loop(0, t // ROW_BLK, sm_step, 0)

    return pl.pallas_call(
        body,
        out_shape=jax.ShapeDtypeStruct((t, v_tot), jnp.float32),
        in_specs=[
            pl.BlockSpec(memory_space=pltpu.VMEM),
            pl.BlockSpec(memory_space=pl.ANY),
        ],
        out_specs=pl.BlockSpec(memory_space=pltpu.VMEM),
        scratch_shapes=[
            pltpu.VMEM((2, d, W_CHUNK), jnp.float32),
            pltpu.SemaphoreType.DMA((2,)),
            pltpu.SemaphoreType.DMA,
            pltpu.SemaphoreType.DMA,
        ],
        compiler_params=pltpu.CompilerParams(
            collective_id=0,
            vmem_limit_bytes=100 * 1024 * 1024,
        ),
    )(x, W)


# device time: 224185 ns/iter; 1.1163x vs baseline; 1.1163x over previous
import jax
import jax.numpy as jnp
from jax import lax
from jax.experimental import pallas as pl
from jax.experimental.pallas import tpu as pltpu

W_CHUNK = 512


def kernel(x, W):
    t, d = x.shape
    _, v_loc = W.shape
    v_tot = 2 * v_loc
    n_chunks = v_loc // W_CHUNK

    def body(x_ref, w_hbm, out_ref, wbuf, wsems, send_sems, recv_sems):
        my_x = lax.axis_index("x")
        my_y = lax.axis_index("y")
        my_z = lax.axis_index("z")
        peer = (1 - my_x, my_y, my_z)

        barrier_sem = pltpu.get_barrier_semaphore()
        pl.semaphore_signal(
            barrier_sem,
            inc=1,
            device_id=peer,
            device_id_type=pl.DeviceIdType.MESH,
        )
        pl.semaphore_wait(barrier_sem, 1)

        def run(xpos):
            base = xpos * v_loc
            pbase = (1 - xpos) * v_loc

            def fetch(j, slot):
                return pltpu.make_async_copy(
                    w_hbm.at[:, pl.ds(j * W_CHUNK, W_CHUNK)],
                    wbuf.at[slot],
                    wsems.at[slot],
                )

            def send_rdma(j):
                return pltpu.make_async_remote_copy(
                    src_ref=out_ref.at[:, pl.ds(base + j * W_CHUNK, W_CHUNK)],
                    dst_ref=out_ref.at[:, pl.ds(base + j * W_CHUNK, W_CHUNK)],
                    send_sem=send_sems.at[j],
                    recv_sem=recv_sems.at[j],
                    device_id=peer,
                    device_id_type=pl.DeviceIdType.MESH,
                )

            def recv_rdma(j):
                return pltpu.make_async_remote_copy(
                    src_ref=out_ref.at[:, pl.ds(pbase + j * W_CHUNK, W_CHUNK)],
                    dst_ref=out_ref.at[:, pl.ds(pbase + j * W_CHUNK, W_CHUNK)],
                    send_sem=send_sems.at[j],
                    recv_sem=recv_sems.at[j],
                    device_id=peer,
                    device_id_type=pl.DeviceIdType.MESH,
                )

            fetch(0, 0).start()

            def gemm_step(j, s_acc):
                slot = j % 2

                @pl.when(j + 1 < n_chunks)
                def _():
                    fetch(j + 1, (j + 1) % 2).start()

                fetch(j, slot).wait()
                e = jnp.exp(
                    jnp.dot(
                        x_ref[:, :], wbuf[slot],
                        preferred_element_type=jnp.float32,
                    )
                )
                out_ref[:, pl.ds(base + j * W_CHUNK, W_CHUNK)] = e
                send_rdma(j).start()
                return s_acc + jnp.sum(e, axis=1, keepdims=True)

            s_loc = lax.fori_loop(
                0, n_chunks, gemm_step, jnp.zeros((t, 1), jnp.float32)
            )

            def recv_step(j, s_acc):
                recv_rdma(j).wait_recv()
                pc = out_ref[:, pl.ds(pbase + j * W_CHUNK, W_CHUNK)]
                return s_acc + jnp.sum(pc, axis=1, keepdims=True)

            s_peer = lax.fori_loop(
                0, n_chunks, recv_step, jnp.zeros((t, 1), jnp.float32)
            )

            recip = 1.0 / (s_loc + s_peer)

            def fin_step(j, carry):
                send_rdma(j).wait_send()
                lc = out_ref[:, pl.ds(base + j * W_CHUNK, W_CHUNK)]
                out_ref[:, pl.ds(base + j * W_CHUNK, W_CHUNK)] = lc * recip
                pc = out_ref[:, pl.ds(pbase + j * W_CHUNK, W_CHUNK)]
                out_ref[:, pl.ds(pbase + j * W_CHUNK, W_CHUNK)] = pc * recip
                return carry

            lax.fori_loop(0, n_chunks, fin_step, 0)

        @pl.when(my_x == 0)
        def _():
            run(0)

        @pl.when(my_x == 1)
        def _():
            run(1)

    return pl.pallas_call(
        body,
        out_shape=jax.ShapeDtypeStruct((t, v_tot), jnp.float32),
        in_specs=[
            pl.BlockSpec(memory_space=pltpu.VMEM),
            pl.BlockSpec(memory_space=pl.ANY),
        ],
        out_specs=pl.BlockSpec(memory_space=pltpu.VMEM),
        scratch_shapes=[
            pltpu.VMEM((2, d, W_CHUNK), jnp.float32),
            pltpu.SemaphoreType.DMA((2,)),
            pltpu.SemaphoreType.DMA((n_chunks,)),
            pltpu.SemaphoreType.DMA((n_chunks,)),
        ],
        compiler_params=pltpu.CompilerParams(
            collective_id=0,
            vmem_limit_bytes=100 * 1024 * 1024,
        ),
    )(x, W)


# device time: 134208 ns/iter; 1.8646x vs baseline; 1.6704x over previous
import jax
import jax.numpy as jnp
from jax import lax
from jax.experimental import pallas as pl
from jax.experimental.pallas import tpu as pltpu

W_CHUNK = 512


def kernel(x, W):
    t, d = x.shape
    _, v_loc = W.shape
    v_tot = 2 * v_loc
    n_chunks = v_loc // W_CHUNK

    def body(x_ref, w_hbm, out_ref, wbuf, sbuf, rbuf, wsems, send_sems,
             recv_sems):
        my_x = lax.axis_index("x")
        my_y = lax.axis_index("y")
        my_z = lax.axis_index("z")
        peer = (1 - my_x, my_y, my_z)

        barrier_sem = pltpu.get_barrier_semaphore()
        pl.semaphore_signal(
            barrier_sem,
            inc=1,
            device_id=peer,
            device_id_type=pl.DeviceIdType.MESH,
        )
        pl.semaphore_wait(barrier_sem, 1)

        def fetch(j, slot):
            return pltpu.make_async_copy(
                w_hbm.at[:, pl.ds(j * W_CHUNK, W_CHUNK)],
                wbuf.at[slot],
                wsems.at[slot],
            )

        def chunk_rdma(j):
            return pltpu.make_async_remote_copy(
                src_ref=sbuf.at[:, pl.ds(j * W_CHUNK, W_CHUNK)],
                dst_ref=rbuf.at[:, pl.ds(j * W_CHUNK, W_CHUNK)],
                send_sem=send_sems.at[j],
                recv_sem=recv_sems.at[j],
                device_id=peer,
                device_id_type=pl.DeviceIdType.MESH,
            )

        def run(xpos):
            base = xpos * v_loc
            pbase = (1 - xpos) * v_loc

            fetch(0, 0).start()

            def gemm_step(j, s_acc):
                slot = j % 2

                @pl.when(j + 1 < n_chunks)
                def _():
                    fetch(j + 1, (j + 1) % 2).start()

                fetch(j, slot).wait()
                e = jnp.exp(
                    jnp.dot(
                        x_ref[:, :], wbuf[slot],
                        preferred_element_type=jnp.float32,
                    )
                )
                out_ref[:, pl.ds(base + j * W_CHUNK, W_CHUNK)] = e
                sbuf[:, pl.ds(j * W_CHUNK, W_CHUNK)] = e.astype(jnp.bfloat16)
                chunk_rdma(j).start()
                return s_acc + jnp.sum(e, axis=1, keepdims=True)

            s_loc = lax.fori_loop(
                0, n_chunks, gemm_step, jnp.zeros((t, 1), jnp.float32)
            )

            def recv_step(j, s_acc):
                chunk_rdma(j).wait_recv()
                pc = rbuf[:, pl.ds(j * W_CHUNK, W_CHUNK)].astype(jnp.float32)
                out_ref[:, pl.ds(pbase + j * W_CHUNK, W_CHUNK)] = pc
                return s_acc + jnp.sum(pc, axis=1, keepdims=True)

            s_peer = lax.fori_loop(
                0, n_chunks, recv_step, jnp.zeros((t, 1), jnp.float32)
            )

            recip = 1.0 / (s_loc + s_peer)

            def fin_step(j, carry):
                chunk_rdma(j).wait_send()
                lc = out_ref[:, pl.ds(base + j * W_CHUNK, W_CHUNK)]
                out_ref[:, pl.ds(base + j * W_CHUNK, W_CHUNK)] = lc * recip
                pc = out_ref[:, pl.ds(pbase + j * W_CHUNK, W_CHUNK)]
                out_ref[:, pl.ds(pbase + j * W_CHUNK, W_CHUNK)] = pc * recip
                return carry

            lax.fori_loop(0, n_chunks, fin_step, 0)

        @pl.when(my_x == 0)
        def _():
            run(0)

        @pl.when(my_x == 1)
        def _():
            run(1)

    return pl.pallas_call(
        body,
        out_shape=jax.ShapeDtypeStruct((t, v_tot), jnp.float32),
        in_specs=[
            pl.BlockSpec(memory_space=pltpu.VMEM),
            pl.BlockSpec(memory_space=pl.ANY),
        ],
        out_specs=pl.BlockSpec(memory_space=pltpu.VMEM),
        scratch_shapes=[
            pltpu.VMEM((2, d, W_CHUNK), jnp.float32),
            pltpu.VMEM((t, v_loc), jnp.bfloat16),
            pltpu.VMEM((t, v_loc), jnp.bfloat16),
            pltpu.SemaphoreType.DMA((2,)),
            pltpu.SemaphoreType.DMA((n_chunks,)),
            pltpu.SemaphoreType.DMA((n_chunks,)),
        ],
        compiler_params=pltpu.CompilerParams(
            collective_id=0,
            vmem_limit_bytes=100 * 1024 * 1024,
        ),
    )(x, W)


# device time: 134168 ns/iter; 1.8652x vs baseline; 1.0003x over previous
import jax
import jax.numpy as jnp
from jax import lax
from jax.experimental import pallas as pl
from jax.experimental.pallas import tpu as pltpu

W_CHUNK = 512
FIN_BLK = 2048


def kernel(x, W):
    t, d = x.shape
    _, v_loc = W.shape
    v_tot = 2 * v_loc
    n_chunks = v_loc // W_CHUNK

    def body(x_ref, w_hbm, out_ref, wbuf, sbuf, rbuf, wsems, send_sems,
             recv_sems):
        my_x = lax.axis_index("x")
        my_y = lax.axis_index("y")
        my_z = lax.axis_index("z")
        peer = (1 - my_x, my_y, my_z)

        barrier_sem = pltpu.get_barrier_semaphore()
        pl.semaphore_signal(
            barrier_sem,
            inc=1,
            device_id=peer,
            device_id_type=pl.DeviceIdType.MESH,
        )
        pl.semaphore_wait(barrier_sem, 1)

        def fetch(j, slot):
            return pltpu.make_async_copy(
                w_hbm.at[:, pl.ds(j * W_CHUNK, W_CHUNK)],
                wbuf.at[slot],
                wsems.at[slot],
            )

        def chunk_rdma(j):
            return pltpu.make_async_remote_copy(
                src_ref=sbuf.at[:, pl.ds(j * W_CHUNK, W_CHUNK)],
                dst_ref=rbuf.at[:, pl.ds(j * W_CHUNK, W_CHUNK)],
                send_sem=send_sems.at[j],
                recv_sem=recv_sems.at[j],
                device_id=peer,
                device_id_type=pl.DeviceIdType.MESH,
            )

        def run(xpos):
            base = xpos * v_loc
            pbase = (1 - xpos) * v_loc

            fetch(0, 0).start()

            def gemm_step(j, s_acc):
                slot = j % 2

                @pl.when(j + 1 < n_chunks)
                def _():
                    fetch(j + 1, (j + 1) % 2).start()

                fetch(j, slot).wait()
                e = jnp.exp(
                    jnp.dot(
                        x_ref[:, :], wbuf[slot],
                        preferred_element_type=jnp.float32,
                    )
                )
                out_ref[:, pl.ds(base + j * W_CHUNK, W_CHUNK)] = e
                sbuf[:, pl.ds(j * W_CHUNK, W_CHUNK)] = e.astype(jnp.bfloat16)
                chunk_rdma(j).start()
                return s_acc + jnp.sum(e, axis=1, keepdims=True)

            s_loc = lax.fori_loop(
                0, n_chunks, gemm_step, jnp.zeros((t, 1), jnp.float32)
            )

            def recv_step(j, s_acc):
                chunk_rdma(j).wait_recv()
                pc = rbuf[:, pl.ds(j * W_CHUNK, W_CHUNK)].astype(jnp.float32)
                return s_acc + jnp.sum(pc, axis=1, keepdims=True)

            s_peer = lax.fori_loop(
                0, n_chunks, recv_step, jnp.zeros((t, 1), jnp.float32)
            )

            recip = 1.0 / (s_loc + s_peer)

            def send_drain(j, carry):
                chunk_rdma(j).wait_send()
                return carry

            lax.fori_loop(0, n_chunks, send_drain, 0)

            def fin_step(j, carry):
                lc = out_ref[:, pl.ds(base + j * FIN_BLK, FIN_BLK)]
                out_ref[:, pl.ds(base + j * FIN_BLK, FIN_BLK)] = lc * recip
                pc = rbuf[:, pl.ds(j * FIN_BLK, FIN_BLK)].astype(jnp.float32)
                out_ref[:, pl.ds(pbase + j * FIN_BLK, FIN_BLK)] = pc * recip
                return carry

            lax.fori_loop(0, v_loc // FIN_BLK, fin_step, 0)

        @pl.when(my_x == 0)
        def _():
            run(0)

        @pl.when(my_x == 1)
        def _():
            run(1)

    return pl.pallas_call(
        body,
        out_shape=jax.ShapeDtypeStruct((t, v_tot), jnp.float32),
        in_specs=[
            pl.BlockSpec(memory_space=pltpu.VMEM),
            pl.BlockSpec(memory_space=pl.ANY),
        ],
        out_specs=pl.BlockSpec(memory_space=pltpu.VMEM),
        scratch_shapes=[
            pltpu.VMEM((2, d, W_CHUNK), jnp.float32),
            pltpu.VMEM((t, v_loc), jnp.bfloat16),
            pltpu.VMEM((t, v_loc), jnp.bfloat16),
            pltpu.SemaphoreType.DMA((2,)),
            pltpu.SemaphoreType.DMA((n_chunks,)),
            pltpu.SemaphoreType.DMA((n_chunks,)),
        ],
        compiler_params=pltpu.CompilerParams(
            collective_id=0,
            vmem_limit_bytes=100 * 1024 * 1024,
        ),
    )(x, W)
